# baseline (speedup 1.0000x reference)
import functools

import jax
import jax.numpy as jnp
from jax import lax
from jax.experimental import pallas as pl
from jax.experimental.pallas import tpu as pltpu
from jax.experimental.pallas import tpu_sc as plsc

_NC = 2
_NS = 16
_NW = _NC * _NS
_L = 16


def kernel(input_ids, word_embeddings, position_embeddings):
    B, S = input_ids.shape
    _, H = word_embeddings.shape
    W = S // _NW
    C = 32
    ids_flat = input_ids.reshape(B * S).astype(jnp.int32)

    mesh = plsc.VectorSubcoreMesh(core_axis_name="c", subcore_axis_name="s")

    @functools.partial(
        pl.kernel,
        mesh=mesh,
        out_type=jax.ShapeDtypeStruct((B * S, H), jnp.float32),
        scratch_types=[
            pltpu.VMEM((B * W,), jnp.int32),
            pltpu.VMEM((C, H), jnp.float32),
            pltpu.VMEM((C, H), jnp.float32),
            pltpu.SemaphoreType.DMA,
        ],
    )
    def embed(ids_hbm, we_hbm, pe_hbm, out_hbm, idx_v, pos_v, rows_v, sem):
        wid = lax.axis_index("s") * _NC + lax.axis_index("c")
        s_base = wid * W
        for b in range(B):
            pltpu.sync_copy(ids_hbm.at[pl.ds(b * S + s_base, W)],
                            idx_v.at[pl.ds(b * W, W)])

        @pl.loop(0, W, step=C)
        def _chunk(c0):
            pltpu.sync_copy(pe_hbm.at[pl.ds(s_base + c0, C)], pos_v)

            @pl.loop(0, B)
            def _batch(b):
                pltpu.async_copy(we_hbm.at[idx_v.at[pl.ds(b * W + c0, C)]],
                                 rows_v, sem).wait()

                @pl.loop(0, C)
                def _row(r):
                    @pl.loop(0, H, step=_L)
                    def _col(j):
                        rows_v.at[r, pl.ds(j, _L)][...] = (
                            rows_v.at[r, pl.ds(j, _L)][...]
                            + pos_v.at[r, pl.ds(j, _L)][...])

                pltpu.sync_copy(rows_v,
                                out_hbm.at[pl.ds(b * S + s_base + c0, C)])

    out = embed(ids_flat, word_embeddings, position_embeddings)
    return out.reshape(B, S, H)


# speedup vs baseline: 1.0597x; 1.0597x over previous
import functools

import jax
import jax.numpy as jnp
from jax import lax
from jax.experimental import pallas as pl
from jax.experimental.pallas import tpu as pltpu
from jax.experimental.pallas import tpu_sc as plsc

_NC = 2
_NS = 16
_NW = _NC * _NS
_L = 16
_C = 16
_NBUF = 3


def kernel(input_ids, word_embeddings, position_embeddings):
    B, S = input_ids.shape
    _, H = word_embeddings.shape
    W = S // _NW
    ids_flat = input_ids.reshape(B * S).astype(jnp.int32)

    mesh = plsc.VectorSubcoreMesh(core_axis_name="c", subcore_axis_name="s")

    @functools.partial(
        pl.kernel,
        mesh=mesh,
        out_type=jax.ShapeDtypeStruct((B * S, H), jnp.float32),
        scratch_types=[
            pltpu.VMEM((B * W,), jnp.int32),
            pltpu.VMEM((W, H), jnp.float32),
            pltpu.VMEM((_C, H), jnp.float32),
            pltpu.VMEM((_C, H), jnp.float32),
            pltpu.VMEM((_C, H), jnp.float32),
            pltpu.SemaphoreType.DMA,
            pltpu.SemaphoreType.DMA,
            pltpu.SemaphoreType.DMA,
            pltpu.SemaphoreType.DMA,
            pltpu.SemaphoreType.DMA,
            pltpu.SemaphoreType.DMA,
        ],
    )
    def embed(ids_hbm, we_hbm, pe_hbm, out_hbm, idx_v, pos_v,
              rows0, rows1, rows2, g0, g1, g2, w0, w1, w2):
        wid = lax.axis_index("s") * _NC + lax.axis_index("c")
        s_base = wid * W
        for b in range(B):
            pltpu.sync_copy(ids_hbm.at[pl.ds(b * S + s_base, W)],
                            idx_v.at[pl.ds(b * W, W)])
        pltpu.sync_copy(pe_hbm.at[pl.ds(s_base, W)], pos_v)

        bufs = (rows0, rows1, rows2)
        gsems = (g0, g1, g2)
        wsems = (w0, w1, w2)
        n_items = (W // _C) * B

        def item_params(i):
            return i % B, (i // B) * _C

        def fire_gather(i):
            b, s_off = item_params(i)
            return pltpu.async_copy(
                we_hbm.at[idx_v.at[pl.ds(b * W + s_off, _C)]],
                bufs[i % _NBUF], gsems[i % _NBUF])

        def add_pos(buf, s_off):
            @pl.loop(0, _C)
            def _row(r):
                @pl.loop(0, H, step=_L)
                def _col(c):
                    buf.at[r, pl.ds(c, _L)][...] = (
                        buf.at[r, pl.ds(c, _L)][...]
                        + pos_v.at[s_off + r, pl.ds(c, _L)][...])

        g_cps = {0: fire_gather(0)}
        if n_items > 1:
            g_cps[1] = fire_gather(1)
        w_cps = {}
        for i in range(n_items):
            k = i % _NBUF
            b, s_off = item_params(i)
            j = i + 2
            if j < n_items:
                if j - _NBUF in w_cps:
                    w_cps[j - _NBUF].wait()
                g_cps[j] = fire_gather(j)
            g_cps[i].wait()
            add_pos(bufs[k], s_off)
            w_cps[i] = pltpu.async_copy(
                bufs[k], out_hbm.at[pl.ds(b * S + s_base + s_off, _C)],
                wsems[k])
        for i in range(max(0, n_items - _NBUF), n_items):
            w_cps[i].wait()

    out = embed(ids_flat, word_embeddings, position_embeddings)
    return out.reshape(B, S, H)


# speedup vs baseline: 2.1612x; 2.0394x over previous
import functools

import jax
import jax.numpy as jnp
from jax import lax
from jax.experimental import pallas as pl
from jax.experimental.pallas import tpu as pltpu
from jax.experimental.pallas import tpu_sc as plsc

_NC = 2
_NS = 16
_NW = _NC * _NS
_L = 16
_C = 16
_NBUF = 3


def kernel(input_ids, word_embeddings, position_embeddings):
    B, S = input_ids.shape
    _, H = word_embeddings.shape
    W = S // _NW
    ids_flat = input_ids.reshape(B * S).astype(jnp.int32)

    mesh = plsc.VectorSubcoreMesh(core_axis_name="c", subcore_axis_name="s")

    @functools.partial(
        pl.kernel,
        mesh=mesh,
        out_type=jax.ShapeDtypeStruct((B * S, H), jnp.float32),
        scratch_types=[
            pltpu.VMEM((B * W,), jnp.int32),
            pltpu.VMEM((W, H), jnp.float32),
            pltpu.VMEM((_C, H), jnp.float32),
            pltpu.VMEM((_C, H), jnp.float32),
            pltpu.VMEM((_C, H), jnp.float32),
            pltpu.SemaphoreType.DMA,
            pltpu.SemaphoreType.DMA,
            pltpu.SemaphoreType.DMA,
            pltpu.SemaphoreType.DMA,
            pltpu.SemaphoreType.DMA,
            pltpu.SemaphoreType.DMA,
        ],
    )
    def embed(ids_hbm, we_hbm, pe_hbm, out_hbm, idx_v, pos_v,
              rows0, rows1, rows2, g0, g1, g2, w0, w1, w2):
        wid = lax.axis_index("s") * _NC + lax.axis_index("c")
        s_base = wid * W
        for b in range(B):
            pltpu.sync_copy(ids_hbm.at[pl.ds(b * S + s_base, W)],
                            idx_v.at[pl.ds(b * W, W)])
        pltpu.sync_copy(pe_hbm.at[pl.ds(s_base, W)], pos_v)

        bufs = (rows0, rows1, rows2)
        gsems = (g0, g1, g2)
        wsems = (w0, w1, w2)
        n_items = (W // _C) * B

        def item_params(i):
            return i % B, (i // B) * _C

        def fire_gather(i):
            b, s_off = item_params(i)
            return pltpu.async_copy(
                we_hbm.at[idx_v.at[pl.ds(b * W + s_off, _C)]],
                bufs[i % _NBUF], gsems[i % _NBUF])

        def add_pos(buf, s_off):
            @pl.loop(0, _C)
            def _row(r):
                @plsc.parallel_loop(0, H, step=_L, unroll=8)
                def _col(c):
                    buf.at[r, pl.ds(c, _L)][...] = (
                        buf.at[r, pl.ds(c, _L)][...]
                        + pos_v.at[s_off + r, pl.ds(c, _L)][...])

        g_cps = {0: fire_gather(0)}
        if n_items > 1:
            g_cps[1] = fire_gather(1)
        w_cps = {}
        for i in range(n_items):
            k = i % _NBUF
            b, s_off = item_params(i)
            j = i + 2
            if j < n_items:
                if j - _NBUF in w_cps:
                    w_cps[j - _NBUF].wait()
                g_cps[j] = fire_gather(j)
            g_cps[i].wait()
            add_pos(bufs[k], s_off)
            w_cps[i] = pltpu.async_copy(
                bufs[k], out_hbm.at[pl.ds(b * S + s_base + s_off, _C)],
                wsems[k])
        for i in range(max(0, n_items - _NBUF), n_items):
            w_cps[i].wait()

    out = embed(ids_flat, word_embeddings, position_embeddings)
    return out.reshape(B, S, H)


# speedup vs baseline: 2.5143x; 1.1634x over previous
import functools

import jax
import jax.numpy as jnp
from jax import lax
from jax.experimental import pallas as pl
from jax.experimental.pallas import tpu as pltpu
from jax.experimental.pallas import tpu_sc as plsc

_NC = 2
_NS = 16
_NW = _NC * _NS
_L = 16
_C = 16
_NBUF = 4
_LOOKAHEAD = 2


def kernel(input_ids, word_embeddings, position_embeddings):
    B, S = input_ids.shape
    _, H = word_embeddings.shape
    W = S // _NW
    n_chunks = W // _C
    ids_flat = input_ids.reshape(B * S).astype(jnp.int32)

    mesh = plsc.VectorSubcoreMesh(core_axis_name="c", subcore_axis_name="s")

    @functools.partial(
        pl.kernel,
        mesh=mesh,
        out_type=jax.ShapeDtypeStruct((B * S, H), jnp.float32),
        scratch_types=(
            [pltpu.VMEM((B * W,), jnp.int32)]
            + [pltpu.VMEM((_C, H), jnp.float32) for _ in range(_NBUF + 2)]
            + [pltpu.SemaphoreType.DMA for _ in range(2 * _NBUF + 2)]
        ),
    )
    def embed(ids_hbm, we_hbm, pe_hbm, out_hbm, idx_v, *bufs_and_sems):
        rbufs = bufs_and_sems[:_NBUF]
        pbufs = bufs_and_sems[_NBUF:_NBUF + 2]
        gsems = bufs_and_sems[_NBUF + 2:2 * _NBUF + 2]
        wsems = bufs_and_sems[2 * _NBUF + 2:3 * _NBUF + 2]
        psems = bufs_and_sems[3 * _NBUF + 2:]

        wid = lax.axis_index("s") * _NC + lax.axis_index("c")
        s_base = wid * W
        for b in range(B):
            pltpu.sync_copy(ids_hbm.at[pl.ds(b * S + s_base, W)],
                            idx_v.at[pl.ds(b * W, W)])

        n_items = n_chunks * B

        def item_params(i):
            return i // B, i % B

        def fire_gather(i):
            sc, b = item_params(i)
            return pltpu.async_copy(
                we_hbm.at[idx_v.at[pl.ds(b * W + sc * _C, _C)]],
                rbufs[i % _NBUF], gsems[i % _NBUF])

        def fire_pos(sc):
            return pltpu.async_copy(
                pe_hbm.at[pl.ds(s_base + sc * _C, _C)],
                pbufs[sc % 2], psems[sc % 2])

        def add_pos(buf, pbuf):
            @pl.loop(0, _C)
            def _row(r):
                @plsc.parallel_loop(0, H, step=_L, unroll=8)
                def _col(c):
                    buf.at[r, pl.ds(c, _L)][...] = (
                        buf.at[r, pl.ds(c, _L)][...]
                        + pbuf.at[r, pl.ds(c, _L)][...])

        p_cps = {0: fire_pos(0)}
        g_cps = {i: fire_gather(i) for i in range(min(_LOOKAHEAD, n_items))}
        w_cps = {}
        w_waited = set()
        for i in range(n_items):
            k = i % _NBUF
            sc, b = item_params(i)
            if b == 0:
                p_cps[sc].wait()
                if sc + 1 < n_chunks:
                    p_cps[sc + 1] = fire_pos(sc + 1)
            j = i + _LOOKAHEAD
            if j < n_items:
                if j - _NBUF in w_cps:
                    w_cps[j - _NBUF].wait()
                    w_waited.add(j - _NBUF)
                g_cps[j] = fire_gather(j)
            g_cps[i].wait()
            add_pos(rbufs[k], pbufs[sc % 2])
            w_cps[i] = pltpu.async_copy(
                rbufs[k], out_hbm.at[pl.ds(b * S + s_base + sc * _C, _C)],
                wsems[k])
        for i in range(n_items):
            if i not in w_waited:
                w_cps[i].wait()

    out = embed(ids_flat, word_embeddings, position_embeddings)
    return out.reshape(B, S, H)


# speedup vs baseline: 2.5603x; 1.0183x over previous
import functools

import jax
import jax.numpy as jnp
from jax import lax
from jax.experimental import pallas as pl
from jax.experimental.pallas import tpu as pltpu
from jax.experimental.pallas import tpu_sc as plsc

_NC = 2
_NS = 16
_NW = _NC * _NS
_L = 16
_C = 16
_NBUF = 5
_LOOKAHEAD = 3


def kernel(input_ids, word_embeddings, position_embeddings):
    B, S = input_ids.shape
    _, H = word_embeddings.shape
    W = S // _NW
    n_chunks = W // _C
    ids = input_ids.astype(jnp.int32)

    mesh = plsc.VectorSubcoreMesh(core_axis_name="c", subcore_axis_name="s")

    @functools.partial(
        pl.kernel,
        mesh=mesh,
        out_type=jax.ShapeDtypeStruct((B * S, H), jnp.float32),
        scratch_types=(
            [pltpu.VMEM((B * W,), jnp.int32)]
            + [pltpu.VMEM((_C, H), jnp.float32) for _ in range(_NBUF + 2)]
            + [pltpu.SemaphoreType.DMA for _ in range(2 * _NBUF + 3)]
        ),
    )
    def embed(ids_hbm, we_hbm, pe_hbm, out_hbm, idx_v, *bufs_and_sems):
        rbufs = bufs_and_sems[:_NBUF]
        pbufs = bufs_and_sems[_NBUF:_NBUF + 2]
        sems = bufs_and_sems[_NBUF + 2:]
        gsems = sems[:_NBUF]
        wsems = sems[_NBUF:2 * _NBUF]
        psems = sems[2 * _NBUF:2 * _NBUF + 2]
        isem = sems[2 * _NBUF + 2]

        wid = lax.axis_index("s") * _NC + lax.axis_index("c")
        s_base = wid * W

        i_cps = [
            pltpu.async_copy(ids_hbm.at[b].at[pl.ds(s_base, W)],
                             idx_v.at[pl.ds(b * W, W)], isem)
            for b in range(B)
        ]
        n_items = n_chunks * B

        def item_params(i):
            return i // B, i % B

        def fire_gather(i):
            sc, b = item_params(i)
            return pltpu.async_copy(
                we_hbm.at[idx_v.at[pl.ds(b * W + sc * _C, _C)]],
                rbufs[i % _NBUF], gsems[i % _NBUF])

        def fire_pos(sc):
            return pltpu.async_copy(
                pe_hbm.at[pl.ds(s_base + sc * _C, _C)],
                pbufs[sc % 2], psems[sc % 2])

        def add_pos(buf, pbuf):
            @pl.loop(0, _C)
            def _row(r):
                @plsc.parallel_loop(0, H, step=_L, unroll=8)
                def _col(c):
                    buf.at[r, pl.ds(c, _L)][...] = (
                        buf.at[r, pl.ds(c, _L)][...]
                        + pbuf.at[r, pl.ds(c, _L)][...])

        p_cps = {0: fire_pos(0)}
        for cp in i_cps:
            cp.wait()
        g_cps = {i: fire_gather(i) for i in range(min(_LOOKAHEAD, n_items))}
        w_cps = {}
        w_waited = set()
        for i in range(n_items):
            k = i % _NBUF
            sc, b = item_params(i)
            if b == 0:
                p_cps[sc].wait()
                if sc + 1 < n_chunks:
                    p_cps[sc + 1] = fire_pos(sc + 1)
            j = i + _LOOKAHEAD
            if j < n_items:
                if j - _NBUF in w_cps:
                    w_cps[j - _NBUF].wait()
                    w_waited.add(j - _NBUF)
                g_cps[j] = fire_gather(j)
            g_cps[i].wait()
            add_pos(rbufs[k], pbufs[sc % 2])
            w_cps[i] = pltpu.async_copy(
                rbufs[k], out_hbm.at[pl.ds(b * S + s_base + sc * _C, _C)],
                wsems[k])
        for i in range(n_items):
            if i not in w_waited:
                w_cps[i].wait()

    out = embed(ids, word_embeddings, position_embeddings)
    return out.reshape(B, S, H)
